# initial kernel scaffold (unmeasured)
import jax
import jax.numpy as jnp
from jax import lax
from jax.experimental import pallas as pl
from jax.experimental.pallas import tpu as pltpu

N_DEV = 4
HQ_LOC = 8
SQ = 1024
SKV_EFF = 1024
DH = 128
DM = 1024
BLK = 64
SCALE = 0.08838834764831843


def kernel(x, Wq, K_ext, V_ext, Wo):
    bf16 = jnp.bfloat16
    xb = x[0].astype(bf16)
    Wqb = Wq.astype(bf16)
    Kb = jnp.transpose(K_ext[0], (1, 0, 2)).astype(bf16)
    Vb = jnp.transpose(V_ext[0], (1, 0, 2)).astype(bf16)
    Wob = Wo.astype(bf16)

    def body(x_ref, wq_ref, k_ref, v_ref, wo_ref, out_ref,
             q_scr, k_work, v_work, ctx_scr, acc_scr, comm,
             kv_send_sems, kv_recv_sems, ring_send_sems, ring_recv_sems,
             exit_sem):
        my = lax.axis_index("i")
        right = lax.rem(my + 1, N_DEV)

        def kv_rdma(j, t):
            src = k_ref if t == 0 else v_ref
            dst = k_work if t == 0 else v_work
            return pltpu.make_async_remote_copy(
                src_ref=src.at[pl.ds(HQ_LOC * j, HQ_LOC)],
                dst_ref=dst,
                send_sem=kv_send_sems.at[2 * (j - 1) + t],
                recv_sem=kv_recv_sems.at[t],
                device_id=(j,),
                device_id_type=pl.DeviceIdType.MESH,
            )

        barrier = pltpu.get_barrier_semaphore()
        for k in range(1, N_DEV):
            pl.semaphore_signal(
                barrier, inc=1,
                device_id=(lax.rem(my + k, N_DEV),),
                device_id_type=pl.DeviceIdType.MESH,
            )
        pl.semaphore_wait(barrier, N_DEV - 1)

        @pl.when(my == 0)
        def _():
            for j in range(1, N_DEV):
                for t in range(2):
                    kv_rdma(j, t).start()

        q_scr[...] = jnp.dot(
            x_ref[...], wq_ref[...], preferred_element_type=jnp.float32
        ).astype(bf16)

        @pl.when(my == 0)
        def _():
            k_work[...] = k_ref[0:HQ_LOC]
            v_work[...] = v_ref[0:HQ_LOC]

        @pl.when(my != 0)
        def _():
            kv_rdma(1, 0).wait_recv()
            kv_rdma(1, 1).wait_recv()

        rows = lax.broadcasted_iota(jnp.int32, (SQ, SKV_EFF), 0)
        cols = lax.broadcasted_iota(jnp.int32, (SQ, SKV_EFF), 1)
        mask = (cols // BLK) <= (rows // BLK)
        for h in range(HQ_LOC):
            q = q_scr[:, h * DH:(h + 1) * DH]
            kh = k_work[h]
            s = lax.dot_general(
                q, kh, (((1,), (1,)), ((), ())),
                preferred_element_type=jnp.float32,
            ) * SCALE
            s = jnp.where(mask, s, jnp.float32(-1e9))
            m = jnp.max(s, axis=1, keepdims=True)
            w = jnp.exp(s - m)
            d = jnp.sum(w, axis=1, keepdims=True)
            wn = (w / d).astype(bf16)
            ctx = jnp.dot(wn, v_work[h], preferred_element_type=jnp.float32)
            ctx_scr[:, h * DH:(h + 1) * DH] = ctx.astype(bf16)

        acc_scr[...] = jnp.dot(
            ctx_scr[...], wo_ref[...], preferred_element_type=jnp.float32
        )

        @pl.when(my == 0)
        def _():
            for j in range(1, N_DEV):
                for t in range(2):
                    kv_rdma(j, t).wait_send()

        comm[0, :, :] = acc_scr[...].astype(bf16)
        for hop in range(N_DEV - 1):
            s_slot = hop % 2
            r_slot = (hop + 1) % 2
            rdma = pltpu.make_async_remote_copy(
                src_ref=comm.at[s_slot],
                dst_ref=comm.at[r_slot],
                send_sem=ring_send_sems.at[s_slot],
                recv_sem=ring_recv_sems.at[r_slot],
                device_id=(right,),
                device_id_type=pl.DeviceIdType.MESH,
            )
            rdma.start()
            rdma.wait()
            acc_scr[...] = acc_scr[...] + comm[r_slot].astype(jnp.float32)

        out_ref[0, :, :] = acc_scr[...]

        for k in range(1, N_DEV):
            pl.semaphore_signal(
                exit_sem, inc=1,
                device_id=(lax.rem(my + k, N_DEV),),
                device_id_type=pl.DeviceIdType.MESH,
            )
        pl.semaphore_wait(exit_sem, N_DEV - 1)

    return pl.pallas_call(
        body,
        out_shape=jax.ShapeDtypeStruct((1, SQ, DM), jnp.float32),
        in_specs=[pl.BlockSpec(memory_space=pltpu.VMEM)] * 5,
        out_specs=pl.BlockSpec(memory_space=pltpu.VMEM),
        scratch_shapes=[
            pltpu.VMEM((SQ, HQ_LOC * DH), bf16),
            pltpu.VMEM((HQ_LOC, SKV_EFF, DH), bf16),
            pltpu.VMEM((HQ_LOC, SKV_EFF, DH), bf16),
            pltpu.VMEM((SQ, HQ_LOC * DH), bf16),
            pltpu.VMEM((SQ, DM), jnp.float32),
            pltpu.VMEM((2, SQ, DM), bf16),
            pltpu.SemaphoreType.DMA((6,)),
            pltpu.SemaphoreType.DMA((2,)),
            pltpu.SemaphoreType.DMA((2,)),
            pltpu.SemaphoreType.DMA((2,)),
            pltpu.SemaphoreType.REGULAR,
        ],
        compiler_params=pltpu.CompilerParams(collective_id=0),
    )(xb, Wqb, Kb, Vb, Wob)


# baseline (device time: 160432 ns/iter reference)
import jax
import jax.numpy as jnp
from jax import lax
from jax.experimental import pallas as pl
from jax.experimental.pallas import tpu as pltpu

N_DEV = 4
HQ_LOC = 8
HALF = HQ_LOC // 2
SQ = 1024
SKV_EFF = 1024
DH = 128
DM = 1024
BLK = 64
SCALE = 0.08838834764831843


def kernel(x, Wq, K_ext, V_ext, Wo):
    bf16 = jnp.bfloat16
    f32 = jnp.float32
    xb = x[0].astype(bf16)
    Wqb = Wq.astype(bf16)
    Wob = Wo.astype(bf16)

    def quant(t):
        th = jnp.transpose(t[0], (1, 0, 2)).astype(f32)
        s = jnp.max(jnp.abs(th), axis=2, keepdims=True) / 127.0 + 1e-12
        q = jnp.round(th / s).astype(jnp.int8)
        return q, s[..., 0]

    Kq, Ks = quant(K_ext)
    Vq, Vs = quant(V_ext)

    def body(x_ref, wq_ref, kq_ref, ks_ref, vq_ref, vs_ref, wo_ref, out_ref,
             q_scr, k_work, ks_work, v_work, vs_work,
             k_rel, ks_rel, v_rel, vs_rel,
             ctx_scr, acc_scr, comm, comm_l,
             kv_send_sems, kv_recv_sems, rel_recv_sems, fwd_send_sems,
             p2_recv_sems, ring_send_sems, ring_recv_sems,
             ring_l_send_sems, ring_l_recv_sems,
             ring_barrier_sem, exit_sem):
        my = lax.axis_index("i")
        left = lax.rem(my + N_DEV - 1, N_DEV)
        right = lax.rem(my + 1, N_DEV)

        TENSORS = [
            (kq_ref, k_work, k_rel),
            (ks_ref, ks_work, ks_rel),
            (vq_ref, v_work, v_rel),
            (vs_ref, vs_work, vs_rel),
        ]

        def rdma0(t, lo, n, dst, send_i, recv_sem, j):
            src = TENSORS[t][0]
            return pltpu.make_async_remote_copy(
                src_ref=src.at[pl.ds(lo, n)],
                dst_ref=dst,
                send_sem=kv_send_sems.at[send_i],
                recv_sem=recv_sem,
                device_id=(j,),
                device_id_type=pl.DeviceIdType.MESH,
            )

        def build_dev0():
            res = []
            idx = 0
            for j, off in ((1, 0), (3, HALF)):
                for t in range(4):
                    res.append(rdma0(t, 2 * HQ_LOC + off, HALF,
                                     TENSORS[t][2], idx,
                                     rel_recv_sems.at[t], j))
                    idx += 1
            for j, lo in ((1, HQ_LOC), (3, 3 * HQ_LOC)):
                for t in range(4):
                    res.append(rdma0(t, lo, HQ_LOC,
                                     TENSORS[t][1], idx,
                                     kv_recv_sems.at[t], j))
                    idx += 1
            return res

        def fwd_rdma(t, off, recv_slot):
            return pltpu.make_async_remote_copy(
                src_ref=TENSORS[t][2],
                dst_ref=TENSORS[t][1].at[pl.ds(off, HALF)],
                send_sem=fwd_send_sems.at[t],
                recv_sem=p2_recv_sems.at[recv_slot],
                device_id=(2,),
                device_id_type=pl.DeviceIdType.MESH,
            )

        def own_recv_rdma(t):
            return pltpu.make_async_remote_copy(
                src_ref=TENSORS[t][0].at[pl.ds(0, HQ_LOC)],
                dst_ref=TENSORS[t][1],
                send_sem=kv_send_sems.at[t],
                recv_sem=kv_recv_sems.at[t],
                device_id=(0,),
                device_id_type=pl.DeviceIdType.MESH,
            )

        def rel_recv_rdma(t):
            return pltpu.make_async_remote_copy(
                src_ref=TENSORS[t][0].at[pl.ds(0, HALF)],
                dst_ref=TENSORS[t][2],
                send_sem=kv_send_sems.at[t],
                recv_sem=rel_recv_sems.at[t],
                device_id=(0,),
                device_id_type=pl.DeviceIdType.MESH,
            )

        barrier = pltpu.get_barrier_semaphore()
        for k in range(1, N_DEV):
            pl.semaphore_signal(
                barrier, inc=1,
                device_id=(lax.rem(my + k, N_DEV),),
                device_id_type=pl.DeviceIdType.MESH,
            )
        pl.semaphore_wait(barrier, N_DEV - 1)

        @pl.when(my == 0)
        def _():
            for r in build_dev0():
                r.start()

        @pl.when(my == 1)
        def _():
            for t in range(4):
                rel_recv_rdma(t).wait_recv()
                fwd_rdma(t, 0, t).start()

        @pl.when(my == 3)
        def _():
            for t in range(4):
                rel_recv_rdma(t).wait_recv()
                fwd_rdma(t, HALF, 4 + t).start()

        q_scr[...] = jnp.dot(
            x_ref[...], wq_ref[...], preferred_element_type=f32
        ).astype(bf16)

        @pl.when(my == 0)
        def _():
            k_work[...] = kq_ref[0:HQ_LOC]
            ks_work[...] = ks_ref[0:HQ_LOC]
            v_work[...] = vq_ref[0:HQ_LOC]
            vs_work[...] = vs_ref[0:HQ_LOC]

        @pl.when(jnp.logical_or(my == 1, my == 3))
        def _():
            for t in range(4):
                own_recv_rdma(t).wait_recv()

        @pl.when(my == 2)
        def _():
            for t in range(4):
                fwd_rdma(t, 0, t).wait_recv()
                fwd_rdma(t, HALF, 4 + t).wait_recv()

        rows = lax.broadcasted_iota(jnp.int32, (SQ, SKV_EFF), 0)
        cols = lax.broadcasted_iota(jnp.int32, (SQ, SKV_EFF), 1)
        mask = (cols // BLK) <= (rows // BLK)
        for h in range(HQ_LOC):
            q = q_scr[:, h * DH:(h + 1) * DH]
            kh = k_work[h].astype(bf16)
            s = lax.dot_general(
                q, kh, (((1,), (1,)), ((), ())),
                preferred_element_type=f32,
            ) * (ks_work[h:h + 1, :] * SCALE)
            s = jnp.where(mask, s, jnp.float32(-1e9))
            m = jnp.max(s, axis=1, keepdims=True)
            w = jnp.exp(s - m)
            dnm = jnp.sum(w, axis=1, keepdims=True)
            wn = ((w / dnm) * vs_work[h:h + 1, :]).astype(bf16)
            ctx = jnp.dot(wn, v_work[h].astype(bf16),
                          preferred_element_type=f32)
            ctx_scr[:, h * DH:(h + 1) * DH] = ctx.astype(bf16)

        acc_scr[...] = jnp.dot(
            ctx_scr[...], wo_ref[...], preferred_element_type=f32
        )

        @pl.when(my == 0)
        def _():
            for r in build_dev0():
                r.wait_send()

        @pl.when(jnp.logical_or(my == 1, my == 3))
        def _():
            for t in range(4):
                fwd_rdma(t, 0, t).wait_send()

        comm[0, :, :] = acc_scr[0:SQ // 2, :].astype(bf16)
        comm_l[0, :, :] = acc_scr[SQ // 2:SQ, :].astype(bf16)

        for nbr in (left, right):
            pl.semaphore_signal(
                ring_barrier_sem, inc=1,
                device_id=(nbr,), device_id_type=pl.DeviceIdType.MESH,
            )
        pl.semaphore_wait(ring_barrier_sem, 2)

        for hop in range(N_DEV - 1):
            s_slot = hop % 2
            r_slot = (hop + 1) % 2
            rdma_r = pltpu.make_async_remote_copy(
                src_ref=comm.at[s_slot],
                dst_ref=comm.at[r_slot],
                send_sem=ring_send_sems.at[s_slot],
                recv_sem=ring_recv_sems.at[r_slot],
                device_id=(right,),
                device_id_type=pl.DeviceIdType.MESH,
            )
            rdma_l = pltpu.make_async_remote_copy(
                src_ref=comm_l.at[s_slot],
                dst_ref=comm_l.at[r_slot],
                send_sem=ring_l_send_sems.at[s_slot],
                recv_sem=ring_l_recv_sems.at[r_slot],
                device_id=(left,),
                device_id_type=pl.DeviceIdType.MESH,
            )
            rdma_r.start()
            rdma_l.start()
            rdma_r.wait()
            rdma_l.wait()
            acc_scr[0:SQ // 2, :] = (
                acc_scr[0:SQ // 2, :] + comm[r_slot].astype(f32))
            acc_scr[SQ // 2:SQ, :] = (
                acc_scr[SQ // 2:SQ, :] + comm_l[r_slot].astype(f32))

        out_ref[0, :, :] = acc_scr[...]

        for k in range(1, N_DEV):
            pl.semaphore_signal(
                exit_sem, inc=1,
                device_id=(lax.rem(my + k, N_DEV),),
                device_id_type=pl.DeviceIdType.MESH,
            )
        pl.semaphore_wait(exit_sem, N_DEV - 1)

    i8 = jnp.int8
    return pl.pallas_call(
        body,
        out_shape=jax.ShapeDtypeStruct((1, SQ, DM), jnp.float32),
        in_specs=[pl.BlockSpec(memory_space=pltpu.VMEM)] * 7,
        out_specs=pl.BlockSpec(memory_space=pltpu.VMEM),
        scratch_shapes=[
            pltpu.VMEM((SQ, HQ_LOC * DH), bf16),
            pltpu.VMEM((HQ_LOC, SKV_EFF, DH), i8),
            pltpu.VMEM((HQ_LOC, SKV_EFF), f32),
            pltpu.VMEM((HQ_LOC, SKV_EFF, DH), i8),
            pltpu.VMEM((HQ_LOC, SKV_EFF), f32),
            pltpu.VMEM((HALF, SKV_EFF, DH), i8),
            pltpu.VMEM((HALF, SKV_EFF), f32),
            pltpu.VMEM((HALF, SKV_EFF, DH), i8),
            pltpu.VMEM((HALF, SKV_EFF), f32),
            pltpu.VMEM((SQ, HQ_LOC * DH), bf16),
            pltpu.VMEM((SQ, DM), f32),
            pltpu.VMEM((2, SQ // 2, DM), bf16),
            pltpu.VMEM((2, SQ // 2, DM), bf16),
            pltpu.SemaphoreType.DMA((16,)),
            pltpu.SemaphoreType.DMA((4,)),
            pltpu.SemaphoreType.DMA((4,)),
            pltpu.SemaphoreType.DMA((4,)),
            pltpu.SemaphoreType.DMA((8,)),
            pltpu.SemaphoreType.DMA((2,)),
            pltpu.SemaphoreType.DMA((2,)),
            pltpu.SemaphoreType.DMA((2,)),
            pltpu.SemaphoreType.DMA((2,)),
            pltpu.SemaphoreType.REGULAR,
            pltpu.SemaphoreType.REGULAR,
        ],
        compiler_params=pltpu.CompilerParams(collective_id=0),
    )(xb, Wqb, Kq, Ks, Vq, Vs, Wob)


# device time: 157677 ns/iter; 1.0175x vs baseline; 1.0175x over previous
import jax
import jax.numpy as jnp
from jax import lax
from jax.experimental import pallas as pl
from jax.experimental.pallas import tpu as pltpu

N_DEV = 4
HQ_LOC = 8
HALF = HQ_LOC // 2
SQ = 1024
SKV_EFF = 1024
DH = 128
DM = 1024
BLK = 64
SCALE = 0.08838834764831843


def kernel(x, Wq, K_ext, V_ext, Wo):
    bf16 = jnp.bfloat16
    f32 = jnp.float32
    xb = x[0].astype(bf16)
    Wqb = Wq.astype(bf16)
    Wob = Wo.astype(bf16)

    def quant(t):
        s = jnp.max(jnp.abs(t[0]), axis=2, keepdims=True) / 127.0 + 1e-12
        q = jnp.round(t[0] / s).astype(jnp.int8)
        return (jnp.transpose(q, (1, 0, 2)),
                jnp.transpose(s[..., 0], (1, 0)).astype(f32))

    Kq, Ks = quant(K_ext)
    Vq, Vs = quant(V_ext)

    def body(x_ref, wq_ref, kq_ref, ks_ref, vq_ref, vs_ref, wo_ref, out_ref,
             q_scr, k_work, ks_work, v_work, vs_work,
             k_rel, ks_rel, v_rel, vs_rel,
             ctx_scr, acc_scr, comm, comm_l,
             kv_send_sems, kv_recv_sems, rel_recv_sems, fwd_send_sems,
             p2_recv_sems, ring_send_sems, ring_recv_sems,
             ring_l_send_sems, ring_l_recv_sems,
             ring_barrier_sem, exit_sem):
        my = lax.axis_index("i")
        left = lax.rem(my + N_DEV - 1, N_DEV)
        right = lax.rem(my + 1, N_DEV)

        TENSORS = [
            (kq_ref, k_work, k_rel),
            (ks_ref, ks_work, ks_rel),
            (vq_ref, v_work, v_rel),
            (vs_ref, vs_work, vs_rel),
        ]

        def rdma0(t, lo, n, dst, send_i, recv_sem, j):
            src = TENSORS[t][0]
            return pltpu.make_async_remote_copy(
                src_ref=src.at[pl.ds(lo, n)],
                dst_ref=dst,
                send_sem=kv_send_sems.at[send_i],
                recv_sem=recv_sem,
                device_id=(j,),
                device_id_type=pl.DeviceIdType.MESH,
            )

        def build_dev0():
            res = []
            idx = 0
            for j, off in ((1, 0), (3, HALF)):
                for t in range(4):
                    res.append(rdma0(t, 2 * HQ_LOC + off, HALF,
                                     TENSORS[t][2], idx,
                                     rel_recv_sems.at[t], j))
                    idx += 1
            for j, lo in ((1, HQ_LOC), (3, 3 * HQ_LOC)):
                for t in range(4):
                    res.append(rdma0(t, lo, HQ_LOC,
                                     TENSORS[t][1], idx,
                                     kv_recv_sems.at[t], j))
                    idx += 1
            return res

        def fwd_rdma(t, off, recv_slot):
            return pltpu.make_async_remote_copy(
                src_ref=TENSORS[t][2],
                dst_ref=TENSORS[t][1].at[pl.ds(off, HALF)],
                send_sem=fwd_send_sems.at[t],
                recv_sem=p2_recv_sems.at[recv_slot],
                device_id=(2,),
                device_id_type=pl.DeviceIdType.MESH,
            )

        def own_recv_rdma(t):
            return pltpu.make_async_remote_copy(
                src_ref=TENSORS[t][0].at[pl.ds(0, HQ_LOC)],
                dst_ref=TENSORS[t][1],
                send_sem=kv_send_sems.at[t],
                recv_sem=kv_recv_sems.at[t],
                device_id=(0,),
                device_id_type=pl.DeviceIdType.MESH,
            )

        def rel_recv_rdma(t):
            return pltpu.make_async_remote_copy(
                src_ref=TENSORS[t][0].at[pl.ds(0, HALF)],
                dst_ref=TENSORS[t][2],
                send_sem=kv_send_sems.at[t],
                recv_sem=rel_recv_sems.at[t],
                device_id=(0,),
                device_id_type=pl.DeviceIdType.MESH,
            )

        barrier = pltpu.get_barrier_semaphore()
        for k in range(1, N_DEV):
            pl.semaphore_signal(
                barrier, inc=1,
                device_id=(lax.rem(my + k, N_DEV),),
                device_id_type=pl.DeviceIdType.MESH,
            )
        pl.semaphore_wait(barrier, N_DEV - 1)

        @pl.when(my == 0)
        def _():
            for r in build_dev0():
                r.start()

        @pl.when(my == 1)
        def _():
            for t in range(4):
                rel_recv_rdma(t).wait_recv()
                fwd_rdma(t, 0, t).start()

        @pl.when(my == 3)
        def _():
            for t in range(4):
                rel_recv_rdma(t).wait_recv()
                fwd_rdma(t, HALF, 4 + t).start()

        q_scr[...] = jnp.dot(
            x_ref[...], wq_ref[...], preferred_element_type=f32
        ).astype(bf16)

        @pl.when(my == 0)
        def _():
            k_work[...] = kq_ref[0:HQ_LOC]
            ks_work[...] = ks_ref[0:HQ_LOC]
            v_work[...] = vq_ref[0:HQ_LOC]
            vs_work[...] = vs_ref[0:HQ_LOC]

        @pl.when(jnp.logical_or(my == 1, my == 3))
        def _():
            for t in range(4):
                own_recv_rdma(t).wait_recv()

        @pl.when(my == 2)
        def _():
            for t in range(4):
                fwd_rdma(t, 0, t).wait_recv()
                fwd_rdma(t, HALF, 4 + t).wait_recv()

        rows = lax.broadcasted_iota(jnp.int32, (SQ, SKV_EFF), 0)
        cols = lax.broadcasted_iota(jnp.int32, (SQ, SKV_EFF), 1)
        mask = (cols // BLK) <= (rows // BLK)
        for h in range(HQ_LOC):
            q = q_scr[:, h * DH:(h + 1) * DH]
            kh = k_work[h].astype(bf16)
            s = lax.dot_general(
                q, kh, (((1,), (1,)), ((), ())),
                preferred_element_type=f32,
            ) * (ks_work[h:h + 1, :] * SCALE)
            s = jnp.where(mask, s, jnp.float32(-1e9))
            m = jnp.max(s, axis=1, keepdims=True)
            w = jnp.exp(s - m)
            dnm = jnp.sum(w, axis=1, keepdims=True)
            wn = ((w / dnm) * vs_work[h:h + 1, :]).astype(bf16)
            ctx = jnp.dot(wn, v_work[h].astype(bf16),
                          preferred_element_type=f32)
            ctx_scr[:, h * DH:(h + 1) * DH] = ctx.astype(bf16)

        acc_scr[...] = jnp.dot(
            ctx_scr[...], wo_ref[...], preferred_element_type=f32
        )

        @pl.when(my == 0)
        def _():
            for r in build_dev0():
                r.wait_send()

        @pl.when(jnp.logical_or(my == 1, my == 3))
        def _():
            for t in range(4):
                fwd_rdma(t, 0, t).wait_send()

        comm[0, :, :] = acc_scr[0:SQ // 2, :].astype(bf16)
        comm_l[0, :, :] = acc_scr[SQ // 2:SQ, :].astype(bf16)

        for nbr in (left, right):
            pl.semaphore_signal(
                ring_barrier_sem, inc=1,
                device_id=(nbr,), device_id_type=pl.DeviceIdType.MESH,
            )
        pl.semaphore_wait(ring_barrier_sem, 2)

        for hop in range(N_DEV - 1):
            s_slot = hop % 2
            r_slot = (hop + 1) % 2
            rdma_r = pltpu.make_async_remote_copy(
                src_ref=comm.at[s_slot],
                dst_ref=comm.at[r_slot],
                send_sem=ring_send_sems.at[s_slot],
                recv_sem=ring_recv_sems.at[r_slot],
                device_id=(right,),
                device_id_type=pl.DeviceIdType.MESH,
            )
            rdma_l = pltpu.make_async_remote_copy(
                src_ref=comm_l.at[s_slot],
                dst_ref=comm_l.at[r_slot],
                send_sem=ring_l_send_sems.at[s_slot],
                recv_sem=ring_l_recv_sems.at[r_slot],
                device_id=(left,),
                device_id_type=pl.DeviceIdType.MESH,
            )
            rdma_r.start()
            rdma_l.start()
            rdma_r.wait()
            rdma_l.wait()
            acc_scr[0:SQ // 2, :] = (
                acc_scr[0:SQ // 2, :] + comm[r_slot].astype(f32))
            acc_scr[SQ // 2:SQ, :] = (
                acc_scr[SQ // 2:SQ, :] + comm_l[r_slot].astype(f32))

        out_ref[0, :, :] = acc_scr[...]

        for k in range(1, N_DEV):
            pl.semaphore_signal(
                exit_sem, inc=1,
                device_id=(lax.rem(my + k, N_DEV),),
                device_id_type=pl.DeviceIdType.MESH,
            )
        pl.semaphore_wait(exit_sem, N_DEV - 1)

    i8 = jnp.int8
    return pl.pallas_call(
        body,
        out_shape=jax.ShapeDtypeStruct((1, SQ, DM), jnp.float32),
        in_specs=[pl.BlockSpec(memory_space=pltpu.VMEM)] * 7,
        out_specs=pl.BlockSpec(memory_space=pltpu.VMEM),
        scratch_shapes=[
            pltpu.VMEM((SQ, HQ_LOC * DH), bf16),
            pltpu.VMEM((HQ_LOC, SKV_EFF, DH), i8),
            pltpu.VMEM((HQ_LOC, SKV_EFF), f32),
            pltpu.VMEM((HQ_LOC, SKV_EFF, DH), i8),
            pltpu.VMEM((HQ_LOC, SKV_EFF), f32),
            pltpu.VMEM((HALF, SKV_EFF, DH), i8),
            pltpu.VMEM((HALF, SKV_EFF), f32),
            pltpu.VMEM((HALF, SKV_EFF, DH), i8),
            pltpu.VMEM((HALF, SKV_EFF), f32),
            pltpu.VMEM((SQ, HQ_LOC * DH), bf16),
            pltpu.VMEM((SQ, DM), f32),
            pltpu.VMEM((2, SQ // 2, DM), bf16),
            pltpu.VMEM((2, SQ // 2, DM), bf16),
            pltpu.SemaphoreType.DMA((16,)),
            pltpu.SemaphoreType.DMA((4,)),
            pltpu.SemaphoreType.DMA((4,)),
            pltpu.SemaphoreType.DMA((4,)),
            pltpu.SemaphoreType.DMA((8,)),
            pltpu.SemaphoreType.DMA((2,)),
            pltpu.SemaphoreType.DMA((2,)),
            pltpu.SemaphoreType.DMA((2,)),
            pltpu.SemaphoreType.DMA((2,)),
            pltpu.SemaphoreType.REGULAR,
            pltpu.SemaphoreType.REGULAR,
        ],
        compiler_params=pltpu.CompilerParams(collective_id=0),
    )(xb, Wqb, Kq, Ks, Vq, Vs, Wob)


# device time: 147850 ns/iter; 1.0851x vs baseline; 1.0665x over previous
import jax
import jax.numpy as jnp
from jax import lax
from jax.experimental import pallas as pl
from jax.experimental.pallas import tpu as pltpu

N_DEV = 4
HQ_LOC = 8
HALF = HQ_LOC // 2
SQ = 1024
SKV_EFF = 1024
DH = 128
DM = 1024
BLK = 64
SCALE = 0.08838834764831843


def kernel(x, Wq, K_ext, V_ext, Wo):
    bf16 = jnp.bfloat16
    f32 = jnp.float32
    xb = x[0]

    def quant(t):
        mx = jnp.max(jnp.abs(t[0]), axis=2, keepdims=True) + 1e-12
        inv = 127.0 / mx
        q = jnp.round(t[0] * inv).astype(jnp.int8)
        return (jnp.transpose(q, (1, 0, 2)),
                jnp.transpose(mx[..., 0] * (1.0 / 127.0), (1, 0)).astype(f32))

    Kq, Ks = quant(K_ext)
    Vq, Vs = quant(V_ext)

    def body(x_ref, wq_ref, kq_ref, ks_ref, vq_ref, vs_ref, wo_ref, out_ref,
             q_scr, k_work, ks_work, v_work, vs_work,
             k_rel, ks_rel, v_rel, vs_rel,
             ctx_scr, acc_scr, comm, comm_l,
             kv_send_sems, kv_recv_sems, rel_recv_sems, fwd_send_sems,
             p2_recv_sems, ring_send_sems, ring_recv_sems,
             ring_l_send_sems, ring_l_recv_sems,
             ring_barrier_sem, exit_sem):
        my = lax.axis_index("i")
        left = lax.rem(my + N_DEV - 1, N_DEV)
        right = lax.rem(my + 1, N_DEV)

        TENSORS = [
            (kq_ref, k_work, k_rel),
            (ks_ref, ks_work, ks_rel),
            (vq_ref, v_work, v_rel),
            (vs_ref, vs_work, vs_rel),
        ]

        def rdma0(t, lo, n, dst, send_i, recv_sem, j):
            src = TENSORS[t][0]
            return pltpu.make_async_remote_copy(
                src_ref=src.at[pl.ds(lo, n)],
                dst_ref=dst,
                send_sem=kv_send_sems.at[send_i],
                recv_sem=recv_sem,
                device_id=(j,),
                device_id_type=pl.DeviceIdType.MESH,
            )

        def build_dev0():
            res = []
            idx = 0
            for j, off in ((1, 0), (3, HALF)):
                for t in range(4):
                    res.append(rdma0(t, 2 * HQ_LOC + off, HALF,
                                     TENSORS[t][2], idx,
                                     rel_recv_sems.at[t], j))
                    idx += 1
            for j, lo in ((1, HQ_LOC), (3, 3 * HQ_LOC)):
                for t in range(4):
                    res.append(rdma0(t, lo, HQ_LOC,
                                     TENSORS[t][1], idx,
                                     kv_recv_sems.at[t], j))
                    idx += 1
            return res

        def fwd_rdma(t, off, recv_slot):
            return pltpu.make_async_remote_copy(
                src_ref=TENSORS[t][2],
                dst_ref=TENSORS[t][1].at[pl.ds(off, HALF)],
                send_sem=fwd_send_sems.at[t],
                recv_sem=p2_recv_sems.at[recv_slot],
                device_id=(2,),
                device_id_type=pl.DeviceIdType.MESH,
            )

        def own_recv_rdma(t):
            return pltpu.make_async_remote_copy(
                src_ref=TENSORS[t][0].at[pl.ds(0, HQ_LOC)],
                dst_ref=TENSORS[t][1],
                send_sem=kv_send_sems.at[t],
                recv_sem=kv_recv_sems.at[t],
                device_id=(0,),
                device_id_type=pl.DeviceIdType.MESH,
            )

        def rel_recv_rdma(t):
            return pltpu.make_async_remote_copy(
                src_ref=TENSORS[t][0].at[pl.ds(0, HALF)],
                dst_ref=TENSORS[t][2],
                send_sem=kv_send_sems.at[t],
                recv_sem=rel_recv_sems.at[t],
                device_id=(0,),
                device_id_type=pl.DeviceIdType.MESH,
            )

        barrier = pltpu.get_barrier_semaphore()
        for k in range(1, N_DEV):
            pl.semaphore_signal(
                barrier, inc=1,
                device_id=(lax.rem(my + k, N_DEV),),
                device_id_type=pl.DeviceIdType.MESH,
            )
        pl.semaphore_wait(barrier, N_DEV - 1)

        @pl.when(my == 0)
        def _():
            for r in build_dev0():
                r.start()

        @pl.when(my == 1)
        def _():
            for t in range(4):
                rel_recv_rdma(t).wait_recv()
                fwd_rdma(t, 0, t).start()

        @pl.when(my == 3)
        def _():
            for t in range(4):
                rel_recv_rdma(t).wait_recv()
                fwd_rdma(t, HALF, 4 + t).start()

        q_scr[...] = jnp.dot(
            x_ref[...].astype(bf16), wq_ref[...].astype(bf16),
            preferred_element_type=f32,
        ).astype(bf16)

        @pl.when(my == 0)
        def _():
            k_work[...] = kq_ref[0:HQ_LOC]
            ks_work[...] = ks_ref[0:HQ_LOC]
            v_work[...] = vq_ref[0:HQ_LOC]
            vs_work[...] = vs_ref[0:HQ_LOC]

        @pl.when(jnp.logical_or(my == 1, my == 3))
        def _():
            for t in range(4):
                own_recv_rdma(t).wait_recv()

        @pl.when(my == 2)
        def _():
            for t in range(4):
                fwd_rdma(t, 0, t).wait_recv()
                fwd_rdma(t, HALF, 4 + t).wait_recv()

        rows = lax.broadcasted_iota(jnp.int32, (SQ, SKV_EFF), 0)
        cols = lax.broadcasted_iota(jnp.int32, (SQ, SKV_EFF), 1)
        mask = (cols // BLK) <= (rows // BLK)
        for h in range(HQ_LOC):
            q = q_scr[:, h * DH:(h + 1) * DH]
            kh = k_work[h].astype(bf16)
            s = lax.dot_general(
                q, kh, (((1,), (1,)), ((), ())),
                preferred_element_type=f32,
            ) * (ks_work[h:h + 1, :] * SCALE)
            s = jnp.where(mask, s, jnp.float32(-1e9))
            m = jnp.max(s, axis=1, keepdims=True)
            w = jnp.exp(s - m)
            dnm = jnp.sum(w, axis=1, keepdims=True)
            wn = (w * jnp.reciprocal(dnm) * vs_work[h:h + 1, :]).astype(bf16)
            ctx = jnp.dot(wn, v_work[h].astype(bf16),
                          preferred_element_type=f32)
            ctx_scr[:, h * DH:(h + 1) * DH] = ctx.astype(bf16)

        acc_scr[...] = jnp.dot(
            ctx_scr[...], wo_ref[...].astype(bf16),
            preferred_element_type=f32,
        )

        @pl.when(my == 0)
        def _():
            for r in build_dev0():
                r.wait_send()

        @pl.when(jnp.logical_or(my == 1, my == 3))
        def _():
            for t in range(4):
                fwd_rdma(t, 0, t).wait_send()

        comm[0, :, :] = acc_scr[0:SQ // 2, :].astype(bf16)
        comm_l[0, :, :] = acc_scr[SQ // 2:SQ, :].astype(bf16)

        for nbr in (left, right):
            pl.semaphore_signal(
                ring_barrier_sem, inc=1,
                device_id=(nbr,), device_id_type=pl.DeviceIdType.MESH,
            )
        pl.semaphore_wait(ring_barrier_sem, 2)

        for hop in range(N_DEV - 1):
            s_slot = hop % 2
            r_slot = (hop + 1) % 2
            rdma_r = pltpu.make_async_remote_copy(
                src_ref=comm.at[s_slot],
                dst_ref=comm.at[r_slot],
                send_sem=ring_send_sems.at[s_slot],
                recv_sem=ring_recv_sems.at[r_slot],
                device_id=(right,),
                device_id_type=pl.DeviceIdType.MESH,
            )
            rdma_l = pltpu.make_async_remote_copy(
                src_ref=comm_l.at[s_slot],
                dst_ref=comm_l.at[r_slot],
                send_sem=ring_l_send_sems.at[s_slot],
                recv_sem=ring_l_recv_sems.at[r_slot],
                device_id=(left,),
                device_id_type=pl.DeviceIdType.MESH,
            )
            rdma_r.start()
            rdma_l.start()
            rdma_r.wait()
            rdma_l.wait()
            acc_scr[0:SQ // 2, :] = (
                acc_scr[0:SQ // 2, :] + comm[r_slot].astype(f32))
            acc_scr[SQ // 2:SQ, :] = (
                acc_scr[SQ // 2:SQ, :] + comm_l[r_slot].astype(f32))

        out_ref[0, :, :] = acc_scr[...]

        for k in range(1, N_DEV):
            pl.semaphore_signal(
                exit_sem, inc=1,
                device_id=(lax.rem(my + k, N_DEV),),
                device_id_type=pl.DeviceIdType.MESH,
            )
        pl.semaphore_wait(exit_sem, N_DEV - 1)

    i8 = jnp.int8
    return pl.pallas_call(
        body,
        out_shape=jax.ShapeDtypeStruct((1, SQ, DM), jnp.float32),
        in_specs=[pl.BlockSpec(memory_space=pltpu.VMEM)] * 7,
        out_specs=pl.BlockSpec(memory_space=pltpu.VMEM),
        scratch_shapes=[
            pltpu.VMEM((SQ, HQ_LOC * DH), bf16),
            pltpu.VMEM((HQ_LOC, SKV_EFF, DH), i8),
            pltpu.VMEM((HQ_LOC, SKV_EFF), f32),
            pltpu.VMEM((HQ_LOC, SKV_EFF, DH), i8),
            pltpu.VMEM((HQ_LOC, SKV_EFF), f32),
            pltpu.VMEM((HALF, SKV_EFF, DH), i8),
            pltpu.VMEM((HALF, SKV_EFF), f32),
            pltpu.VMEM((HALF, SKV_EFF, DH), i8),
            pltpu.VMEM((HALF, SKV_EFF), f32),
            pltpu.VMEM((SQ, HQ_LOC * DH), bf16),
            pltpu.VMEM((SQ, DM), f32),
            pltpu.VMEM((2, SQ // 2, DM), bf16),
            pltpu.VMEM((2, SQ // 2, DM), bf16),
            pltpu.SemaphoreType.DMA((16,)),
            pltpu.SemaphoreType.DMA((4,)),
            pltpu.SemaphoreType.DMA((4,)),
            pltpu.SemaphoreType.DMA((4,)),
            pltpu.SemaphoreType.DMA((8,)),
            pltpu.SemaphoreType.DMA((2,)),
            pltpu.SemaphoreType.DMA((2,)),
            pltpu.SemaphoreType.DMA((2,)),
            pltpu.SemaphoreType.DMA((2,)),
            pltpu.SemaphoreType.REGULAR,
            pltpu.SemaphoreType.REGULAR,
        ],
        compiler_params=pltpu.CompilerParams(collective_id=0),
    )(xb, Wq, Kq, Ks, Vq, Vs, Wo)
